# baseline (speedup 1.0000x reference)
import functools
import math

import jax
import jax.numpy as jnp
from jax import lax
from jax.experimental import pallas as pl
from jax.experimental.pallas import tpu as pltpu
from jax.experimental.pallas import tpu_sc as plsc

_D = 64
_SCALE = math.sqrt(_D)
_IW = 128
_K = 8
_CH = _IW * _K
_L = 16


def _emb_body(nchunks, rows_per_worker, x_hbm, table_hbm, out_hbm,
              idx_v, rows_v, sem):
    ncores = 2
    wid = lax.axis_index("s") * ncores + lax.axis_index("c")
    idx_row0 = wid * (rows_per_worker // _IW)
    out_row0 = wid * rows_per_worker

    @pl.loop(0, nchunks)
    def _chunk(i):
        pltpu.sync_copy(x_hbm.at[pl.ds(idx_row0 + i * _K, _K)], idx_v)
        descs = []
        for k in range(_K):
            descs.append(
                pltpu.async_copy(
                    table_hbm.at[idx_v.at[k]],
                    rows_v.at[pl.ds(k * _IW, _IW)],
                    sem,
                )
            )
        for d in descs:
            d.wait()

        @pl.loop(0, _CH, unroll=2)
        def _scale(j):
            for t in range(_D // _L):
                sl = pl.ds(t * _L, _L)
                rows_v[j, sl] = rows_v[j, sl] * _SCALE

        pltpu.sync_copy(rows_v, out_hbm.at[pl.ds(out_row0 + i * _CH, _CH)])


@functools.partial(jax.jit, static_argnums=())
def _emb_lookup(x_flat2d, table):
    n = x_flat2d.shape[0] * _IW
    mesh = plsc.VectorSubcoreMesh(core_axis_name="c", subcore_axis_name="s")
    nw = mesh.num_cores * mesh.num_subcores
    rows_per_worker = n // nw
    nchunks = rows_per_worker // _CH
    body = functools.partial(_emb_body, nchunks, rows_per_worker)
    return pl.kernel(
        body,
        out_type=jax.ShapeDtypeStruct((n, _D), jnp.float32),
        mesh=mesh,
        scratch_types=[
            pltpu.VMEM((_K, _IW), jnp.int32),
            pltpu.VMEM((_CH, _D), jnp.float32),
            pltpu.SemaphoreType.DMA,
        ],
        compiler_params=pltpu.CompilerParams(use_tc_tiling_on_sc=False),
    )(x_flat2d, table)


def kernel(x, table):
    b, s = x.shape
    x2d = x.astype(jnp.int32).reshape(-1, _IW)
    out = _emb_lookup(x2d, table)
    return out.reshape(b, s, _D)
